# baseline (speedup 1.0000x reference)
import functools

import jax
import jax.numpy as jnp
from jax import lax
from jax.experimental import pallas as pl
from jax.experimental.pallas import tpu as pltpu
from jax.experimental.pallas import tpu_sc as plsc

N_NODES = 10000
IN_FEAT = 128
N_EDGES = 320000

NC, NS, L = 2, 16, 16
NW = NC * NS
E_PER_W = N_EDGES // NW
CHUNK = 128
PAIR = 2
N_BLOCKS = 40
E_PAD = N_BLOCKS * PAIR * CHUNK
ACC_ROWS = 10240
ROWS_PER_SUB = ACC_ROWS // NS
DUMP_ROW = 10016
ZBLK = 128
HE_W = 16


def _sc_aggregate(h, src, dst, he_w, zero_hn):
    mesh = plsc.VectorSubcoreMesh(core_axis_name="c", subcore_axis_name="s")

    @functools.partial(
        pl.kernel,
        out_type=[
            jax.ShapeDtypeStruct((NC, ACC_ROWS, IN_FEAT), jnp.float32),
            jax.ShapeDtypeStruct((NC, ACC_ROWS), jnp.float32),
        ],
        mesh=mesh,
        scratch_types=[
            pltpu.VMEM_SHARED((ACC_ROWS, IN_FEAT), jnp.float32),
            pltpu.VMEM_SHARED((ACC_ROWS,), jnp.float32),
            pltpu.VMEM((PAIR, CHUNK), jnp.int32),
            pltpu.VMEM((PAIR, CHUNK), jnp.int32),
            pltpu.VMEM((PAIR, CHUNK), jnp.float32),
            pltpu.VMEM((CHUNK, IN_FEAT), jnp.float32),
            pltpu.VMEM((CHUNK, IN_FEAT), jnp.float32),
            pltpu.VMEM((ROWS_PER_SUB,), jnp.float32),
            pltpu.SemaphoreType.DMA,
            pltpu.SemaphoreType.DMA,
            pltpu.SemaphoreType.DMA,
        ],
    )
    def agg(h_hbm, src_hbm, dst_hbm, he_hbm, zhn_hbm,
            hn_out_hbm, he_out_hbm,
            hn_acc, he_acc, src_c, dst_c, he_c, rows_a, rows_b, he_stage,
            sem_a, sem_b, sem_he):
        cid = lax.axis_index("c")
        sid = lax.axis_index("s")
        wid = cid * NS + sid
        r0 = sid * ROWS_PER_SUB

        zstage = rows_a.at[pl.ds(0, ZBLK)]
        pltpu.sync_copy(zhn_hbm, zstage)
        for k in range(ROWS_PER_SUB // ZBLK):
            pltpu.sync_copy(zstage, hn_acc.at[pl.ds(r0 + k * ZBLK, ZBLK)])
        for i in range(ROWS_PER_SUB // L):
            he_stage[pl.ds(i * L, L)] = jnp.zeros((L,), jnp.float32)
        pltpu.sync_copy(he_stage, he_acc.at[pl.ds(r0, ROWS_PER_SUB)])

        plsc.subcore_barrier()

        def body(b, carry):
            wb = wid * N_BLOCKS + b
            pltpu.sync_copy(src_hbm.at[wb], src_c)
            pltpu.sync_copy(dst_hbm.at[wb], dst_c)
            pltpu.sync_copy(he_hbm.at[wb], he_c)
            g0 = pltpu.async_copy(h_hbm.at[src_c.at[0]], rows_a, sem_a)
            g0.wait()
            g1 = pltpu.async_copy(h_hbm.at[src_c.at[1]], rows_b, sem_b)
            d0 = pltpu.async_copy(he_c.at[0], he_acc.at[dst_c.at[0]],
                                  sem_he, add=True)
            pltpu.sync_copy(rows_a, hn_acc.at[dst_c.at[0]], add=True)
            g1.wait()
            d1 = pltpu.async_copy(he_c.at[1], he_acc.at[dst_c.at[1]],
                                  sem_he, add=True)
            pltpu.sync_copy(rows_b, hn_acc.at[dst_c.at[1]], add=True)
            d0.wait()
            d1.wait()
            return carry

        lax.fori_loop(0, N_BLOCKS, body, 0)

        plsc.subcore_barrier()

        for k in range(ROWS_PER_SUB // ZBLK):
            rr = r0 + k * ZBLK
            pltpu.sync_copy(hn_acc.at[pl.ds(rr, ZBLK)], zstage)
            pltpu.sync_copy(zstage, hn_out_hbm.at[cid].at[pl.ds(rr, ZBLK)])
        pltpu.sync_copy(he_acc.at[pl.ds(r0, ROWS_PER_SUB)], he_stage)
        pltpu.sync_copy(he_stage, he_out_hbm.at[cid].at[pl.ds(r0, ROWS_PER_SUB)])

    return agg(h, src, dst, he_w, zero_hn)


def _tc_mlp(h, hn_p, he0, he1, W1a, W1b, w1c, b1, W2, b2, W3, b3):
    NB = 10
    BLK = N_NODES // NB

    def body(h_ref, hn_ref, he0_ref, he1_ref,
             W1a_ref, W1b_ref, w1c_ref, b1_ref, W2_ref, b2_ref,
             W3_ref, b3_ref, out_ref):
        hn = hn_ref[0] + hn_ref[1]
        he_col = he0_ref[...] + he1_ref[...]
        x = jnp.dot(h_ref[...], W1a_ref[...], preferred_element_type=jnp.float32)
        x = x + jnp.dot(hn, W1b_ref[...], preferred_element_type=jnp.float32)
        x = jnp.maximum(x + he_col * w1c_ref[...] + b1_ref[...], 0.0)
        x = jnp.maximum(
            jnp.dot(x, W2_ref[...], preferred_element_type=jnp.float32) + b2_ref[...], 0.0)
        out_ref[...] = (
            jnp.dot(x, W3_ref[...], preferred_element_type=jnp.float32) + b3_ref[...])

    full = lambda i: (0, 0)
    rows = lambda i: (i, 0)
    return pl.pallas_call(
        body,
        grid=(NB,),
        in_specs=[
            pl.BlockSpec((BLK, IN_FEAT), rows),
            pl.BlockSpec((NC, BLK, IN_FEAT), lambda i: (0, i, 0)),
            pl.BlockSpec((BLK, 1), rows),
            pl.BlockSpec((BLK, 1), rows),
            pl.BlockSpec((IN_FEAT, IN_FEAT), full),
            pl.BlockSpec((IN_FEAT, IN_FEAT), full),
            pl.BlockSpec((1, IN_FEAT), full),
            pl.BlockSpec((1, IN_FEAT), full),
            pl.BlockSpec((IN_FEAT, IN_FEAT), full),
            pl.BlockSpec((1, IN_FEAT), full),
            pl.BlockSpec((IN_FEAT, 1), full),
            pl.BlockSpec((1, 1), full),
        ],
        out_specs=pl.BlockSpec((BLK, 1), rows),
        out_shape=jax.ShapeDtypeStruct((N_NODES, 1), jnp.float32),
    )(h, hn_p, he0, he1, W1a, W1b, w1c, b1, W2, b2, W3, b3)


def kernel(h, edge_index, he, W1, b1, W2, b2, W3, b3):
    ei = edge_index.astype(jnp.int32)
    pad = E_PAD - E_PER_W
    blocked = (NW * N_BLOCKS, PAIR, CHUNK)
    src = jnp.pad(ei[0].reshape(NW, E_PER_W),
                  ((0, 0), (0, pad))).reshape(blocked)
    dst = jnp.pad(ei[1].reshape(NW, E_PER_W), ((0, 0), (0, pad)),
                  constant_values=DUMP_ROW).reshape(blocked)
    he_w = jnp.pad(he.reshape(NW, E_PER_W),
                   ((0, 0), (0, pad))).reshape(blocked)
    zero_hn = jnp.zeros((ZBLK, IN_FEAT), jnp.float32)
    hn_p, he_p = _sc_aggregate(h, src, dst, he_w, zero_hn)
    he0 = he_p[0, :N_NODES].reshape(N_NODES, 1)
    he1 = he_p[1, :N_NODES].reshape(N_NODES, 1)
    W1a = W1[:IN_FEAT]
    W1b = W1[IN_FEAT:2 * IN_FEAT]
    w1c = W1[2 * IN_FEAT:]
    return _tc_mlp(h, hn_p, he0, he1, W1a, W1b, w1c,
                   b1.reshape(1, IN_FEAT), W2, b2.reshape(1, IN_FEAT),
                   W3, b3.reshape(1, 1))


# speedup vs baseline: 1.1805x; 1.1805x over previous
import functools

import jax
import jax.numpy as jnp
from jax import lax
from jax.experimental import pallas as pl
from jax.experimental.pallas import tpu as pltpu
from jax.experimental.pallas import tpu_sc as plsc

N_NODES = 10000
IN_FEAT = 128
N_EDGES = 320000

NC, NS, L = 2, 16, 16
NW = NC * NS
E_PER_W = N_EDGES // NW
CHUNK = 128
E_PAD = 10112
N_CHUNKS = E_PAD // CHUNK
ACC_ROWS = 10240
ROWS_PER_SUB = ACC_ROWS // NS
DUMP_ROW = 10016
ZBLK = 128
HE_W = 16


def _sc_aggregate(h, src, dst, he_w, zero_hn):
    mesh = plsc.VectorSubcoreMesh(core_axis_name="c", subcore_axis_name="s")

    @functools.partial(
        pl.kernel,
        out_type=[
            jax.ShapeDtypeStruct((NC, ACC_ROWS, IN_FEAT), jnp.float32),
            jax.ShapeDtypeStruct((NC, ACC_ROWS), jnp.float32),
        ],
        mesh=mesh,
        scratch_types=[
            pltpu.VMEM_SHARED((ACC_ROWS, IN_FEAT), jnp.float32),
            pltpu.VMEM_SHARED((ACC_ROWS,), jnp.float32),
            pltpu.VMEM((CHUNK,), jnp.int32),
            pltpu.VMEM((CHUNK,), jnp.int32),
            pltpu.VMEM((CHUNK,), jnp.float32),
            pltpu.VMEM((CHUNK, IN_FEAT), jnp.float32),
            pltpu.VMEM((ROWS_PER_SUB,), jnp.float32),
            pltpu.SemaphoreType.DMA,
            pltpu.SemaphoreType.DMA,
        ],
    )
    def agg(h_hbm, src_hbm, dst_hbm, he_hbm, zhn_hbm,
            hn_out_hbm, he_out_hbm,
            hn_acc, he_acc, src_c, dst_c, he_c, rows_v, he_stage, sem,
            sem_he):
        cid = lax.axis_index("c")
        sid = lax.axis_index("s")
        wid = cid * NS + sid
        r0 = sid * ROWS_PER_SUB

        zstage = rows_v.at[pl.ds(0, ZBLK)]
        pltpu.sync_copy(zhn_hbm, zstage)
        for k in range(ROWS_PER_SUB // ZBLK):
            pltpu.sync_copy(zstage, hn_acc.at[pl.ds(r0 + k * ZBLK, ZBLK)])
        for i in range(ROWS_PER_SUB // L):
            he_stage[pl.ds(i * L, L)] = jnp.zeros((L,), jnp.float32)
        pltpu.sync_copy(he_stage, he_acc.at[pl.ds(r0, ROWS_PER_SUB)])

        plsc.subcore_barrier()

        def body(j, carry):
            e0 = j * CHUNK
            pltpu.sync_copy(src_hbm.at[wid].at[pl.ds(e0, CHUNK)], src_c)
            pltpu.sync_copy(dst_hbm.at[wid].at[pl.ds(e0, CHUNK)], dst_c)
            pltpu.sync_copy(he_hbm.at[wid].at[pl.ds(e0, CHUNK)], he_c)
            pltpu.async_copy(h_hbm.at[src_c], rows_v, sem).wait()
            d_he = pltpu.async_copy(he_c, he_acc.at[dst_c], sem_he, add=True)
            pltpu.sync_copy(rows_v, hn_acc.at[dst_c], add=True)
            d_he.wait()
            return carry

        lax.fori_loop(0, N_CHUNKS, body, 0)

        plsc.subcore_barrier()

        for k in range(ROWS_PER_SUB // ZBLK):
            rr = r0 + k * ZBLK
            pltpu.sync_copy(hn_acc.at[pl.ds(rr, ZBLK)], zstage)
            pltpu.sync_copy(zstage, hn_out_hbm.at[cid].at[pl.ds(rr, ZBLK)])
        pltpu.sync_copy(he_acc.at[pl.ds(r0, ROWS_PER_SUB)], he_stage)
        pltpu.sync_copy(he_stage, he_out_hbm.at[cid].at[pl.ds(r0, ROWS_PER_SUB)])

    return agg(h, src, dst, he_w, zero_hn)


def _tc_mlp(h, hn_p, he0, he1, W1a, W1b, w1c, b1, W2, b2, W3, b3):
    NB = 10
    BLK = N_NODES // NB

    def body(h_ref, hn_ref, he0_ref, he1_ref,
             W1a_ref, W1b_ref, w1c_ref, b1_ref, W2_ref, b2_ref,
             W3_ref, b3_ref, out_ref):
        hn = hn_ref[0] + hn_ref[1]
        he_col = he0_ref[...] + he1_ref[...]
        x = jnp.dot(h_ref[...], W1a_ref[...], preferred_element_type=jnp.float32)
        x = x + jnp.dot(hn, W1b_ref[...], preferred_element_type=jnp.float32)
        x = jnp.maximum(x + he_col * w1c_ref[...] + b1_ref[...], 0.0)
        x = jnp.maximum(
            jnp.dot(x, W2_ref[...], preferred_element_type=jnp.float32) + b2_ref[...], 0.0)
        out_ref[...] = (
            jnp.dot(x, W3_ref[...], preferred_element_type=jnp.float32) + b3_ref[...])

    full = lambda i: (0, 0)
    rows = lambda i: (i, 0)
    return pl.pallas_call(
        body,
        grid=(NB,),
        in_specs=[
            pl.BlockSpec((BLK, IN_FEAT), rows),
            pl.BlockSpec((NC, BLK, IN_FEAT), lambda i: (0, i, 0)),
            pl.BlockSpec((BLK, 1), rows),
            pl.BlockSpec((BLK, 1), rows),
            pl.BlockSpec((IN_FEAT, IN_FEAT), full),
            pl.BlockSpec((IN_FEAT, IN_FEAT), full),
            pl.BlockSpec((1, IN_FEAT), full),
            pl.BlockSpec((1, IN_FEAT), full),
            pl.BlockSpec((IN_FEAT, IN_FEAT), full),
            pl.BlockSpec((1, IN_FEAT), full),
            pl.BlockSpec((IN_FEAT, 1), full),
            pl.BlockSpec((1, 1), full),
        ],
        out_specs=pl.BlockSpec((BLK, 1), rows),
        out_shape=jax.ShapeDtypeStruct((N_NODES, 1), jnp.float32),
    )(h, hn_p, he0, he1, W1a, W1b, w1c, b1, W2, b2, W3, b3)


def kernel(h, edge_index, he, W1, b1, W2, b2, W3, b3):
    ei = edge_index.astype(jnp.int32)
    pad = E_PAD - E_PER_W
    src = jnp.pad(ei[0].reshape(NW, E_PER_W), ((0, 0), (0, pad)))
    dst = jnp.pad(ei[1].reshape(NW, E_PER_W), ((0, 0), (0, pad)),
                  constant_values=DUMP_ROW)
    he_w = jnp.pad(he.reshape(NW, E_PER_W), ((0, 0), (0, pad)))
    zero_hn = jnp.zeros((ZBLK, IN_FEAT), jnp.float32)
    hn_p, he_p = _sc_aggregate(h, src, dst, he_w, zero_hn)
    he0 = he_p[0, :N_NODES].reshape(N_NODES, 1)
    he1 = he_p[1, :N_NODES].reshape(N_NODES, 1)
    W1a = W1[:IN_FEAT]
    W1b = W1[IN_FEAT:2 * IN_FEAT]
    w1c = W1[2 * IN_FEAT:]
    return _tc_mlp(h, hn_p, he0, he1, W1a, W1b, w1c,
                   b1.reshape(1, IN_FEAT), W2, b2.reshape(1, IN_FEAT),
                   W3, b3.reshape(1, 1))


# speedup vs baseline: 1.2402x; 1.0506x over previous
import functools

import jax
import jax.numpy as jnp
from jax import lax
from jax.experimental import pallas as pl
from jax.experimental.pallas import tpu as pltpu
from jax.experimental.pallas import tpu_sc as plsc

N_NODES = 10000
IN_FEAT = 128
N_EDGES = 320000

NC, NS, L = 2, 16, 16
NW = NC * NS
E_PER_W = N_EDGES // NW
CHUNK = 128
E_PAD = 10112
N_CHUNKS = E_PAD // CHUNK
ACC_ROWS = 10240
ROWS_PER_SUB = ACC_ROWS // NS
DUMP_ROW = 10016
ZBLK = 128
HE_W = 16


def _sc_aggregate(h, idx, he_w, zero_hn):
    mesh = plsc.VectorSubcoreMesh(core_axis_name="c", subcore_axis_name="s")

    @functools.partial(
        pl.kernel,
        out_type=[
            jax.ShapeDtypeStruct((NC, ACC_ROWS, IN_FEAT), jnp.float32),
            jax.ShapeDtypeStruct((NC, ACC_ROWS), jnp.float32),
        ],
        mesh=mesh,
        scratch_types=[
            pltpu.VMEM_SHARED((ACC_ROWS, IN_FEAT), jnp.float32),
            pltpu.VMEM_SHARED((ACC_ROWS,), jnp.float32),
            pltpu.VMEM((2 * CHUNK,), jnp.int32),
            pltpu.VMEM((CHUNK,), jnp.float32),
            pltpu.VMEM((CHUNK, IN_FEAT), jnp.float32),
            pltpu.VMEM((ROWS_PER_SUB,), jnp.float32),
            pltpu.SemaphoreType.DMA,
            pltpu.SemaphoreType.DMA,
        ],
    )
    def agg(h_hbm, idx_hbm, he_hbm, zhn_hbm,
            hn_out_hbm, he_out_hbm,
            hn_acc, he_acc, idx_c, he_c, rows_v, he_stage, sem,
            sem_he):
        cid = lax.axis_index("c")
        sid = lax.axis_index("s")
        wid = cid * NS + sid
        r0 = sid * ROWS_PER_SUB

        zstage = rows_v.at[pl.ds(0, ZBLK)]
        pltpu.sync_copy(zhn_hbm, zstage)
        for k in range(ROWS_PER_SUB // ZBLK):
            pltpu.sync_copy(zstage, hn_acc.at[pl.ds(r0 + k * ZBLK, ZBLK)])
        for i in range(ROWS_PER_SUB // L):
            he_stage[pl.ds(i * L, L)] = jnp.zeros((L,), jnp.float32)
        pltpu.sync_copy(he_stage, he_acc.at[pl.ds(r0, ROWS_PER_SUB)])

        plsc.subcore_barrier()

        def body(j, carry):
            pltpu.sync_copy(
                idx_hbm.at[wid].at[pl.ds(j * 2 * CHUNK, 2 * CHUNK)], idx_c)
            pltpu.sync_copy(he_hbm.at[wid].at[pl.ds(j * CHUNK, CHUNK)], he_c)
            dst_c = idx_c.at[pl.ds(0, CHUNK)]
            src_c = idx_c.at[pl.ds(CHUNK, CHUNK)]
            pltpu.async_copy(h_hbm.at[src_c], rows_v, sem).wait()
            d_he = pltpu.async_copy(he_c, he_acc.at[dst_c], sem_he, add=True)
            pltpu.sync_copy(rows_v, hn_acc.at[dst_c], add=True)
            d_he.wait()
            return carry

        lax.fori_loop(0, N_CHUNKS, body, 0)

        plsc.subcore_barrier()

        for k in range(ROWS_PER_SUB // ZBLK):
            rr = r0 + k * ZBLK
            pltpu.sync_copy(hn_acc.at[pl.ds(rr, ZBLK)], zstage)
            pltpu.sync_copy(zstage, hn_out_hbm.at[cid].at[pl.ds(rr, ZBLK)])
        pltpu.sync_copy(he_acc.at[pl.ds(r0, ROWS_PER_SUB)], he_stage)
        pltpu.sync_copy(he_stage, he_out_hbm.at[cid].at[pl.ds(r0, ROWS_PER_SUB)])

    return agg(h, idx, he_w, zero_hn)


def _tc_mlp(h, hn_p, he0, he1, W1a, W1b, w1c, b1, W2, b2, W3, b3):
    NB = 10
    BLK = N_NODES // NB

    def body(h_ref, hn_ref, he0_ref, he1_ref,
             W1a_ref, W1b_ref, w1c_ref, b1_ref, W2_ref, b2_ref,
             W3_ref, b3_ref, out_ref):
        hn = hn_ref[0] + hn_ref[1]
        he_col = he0_ref[...] + he1_ref[...]
        x = jnp.dot(h_ref[...], W1a_ref[...], preferred_element_type=jnp.float32)
        x = x + jnp.dot(hn, W1b_ref[...], preferred_element_type=jnp.float32)
        x = jnp.maximum(x + he_col * w1c_ref[...] + b1_ref[...], 0.0)
        x = jnp.maximum(
            jnp.dot(x, W2_ref[...], preferred_element_type=jnp.float32) + b2_ref[...], 0.0)
        out_ref[...] = (
            jnp.dot(x, W3_ref[...], preferred_element_type=jnp.float32) + b3_ref[...])

    full = lambda i: (0, 0)
    rows = lambda i: (i, 0)
    return pl.pallas_call(
        body,
        grid=(NB,),
        in_specs=[
            pl.BlockSpec((BLK, IN_FEAT), rows),
            pl.BlockSpec((NC, BLK, IN_FEAT), lambda i: (0, i, 0)),
            pl.BlockSpec((BLK, 1), rows),
            pl.BlockSpec((BLK, 1), rows),
            pl.BlockSpec((IN_FEAT, IN_FEAT), full),
            pl.BlockSpec((IN_FEAT, IN_FEAT), full),
            pl.BlockSpec((1, IN_FEAT), full),
            pl.BlockSpec((1, IN_FEAT), full),
            pl.BlockSpec((IN_FEAT, IN_FEAT), full),
            pl.BlockSpec((1, IN_FEAT), full),
            pl.BlockSpec((IN_FEAT, 1), full),
            pl.BlockSpec((1, 1), full),
        ],
        out_specs=pl.BlockSpec((BLK, 1), rows),
        out_shape=jax.ShapeDtypeStruct((N_NODES, 1), jnp.float32),
    )(h, hn_p, he0, he1, W1a, W1b, w1c, b1, W2, b2, W3, b3)


def kernel(h, edge_index, he, W1, b1, W2, b2, W3, b3):
    ei = edge_index.astype(jnp.int32)
    pad = E_PAD - E_PER_W
    src = jnp.pad(ei[0].reshape(NW, E_PER_W), ((0, 0), (0, pad)))
    dst = jnp.pad(ei[1].reshape(NW, E_PER_W), ((0, 0), (0, pad)),
                  constant_values=DUMP_ROW)
    idx = jnp.stack(
        [dst.reshape(NW, N_CHUNKS, CHUNK), src.reshape(NW, N_CHUNKS, CHUNK)],
        axis=2).reshape(NW, 2 * E_PAD)
    he_w = jnp.pad(he.reshape(NW, E_PER_W), ((0, 0), (0, pad)))
    zero_hn = jnp.zeros((ZBLK, IN_FEAT), jnp.float32)
    hn_p, he_p = _sc_aggregate(h, idx, he_w, zero_hn)
    he0 = he_p[0, :N_NODES].reshape(N_NODES, 1)
    he1 = he_p[1, :N_NODES].reshape(N_NODES, 1)
    W1a = W1[:IN_FEAT]
    W1b = W1[IN_FEAT:2 * IN_FEAT]
    w1c = W1[2 * IN_FEAT:]
    return _tc_mlp(h, hn_p, he0, he1, W1a, W1b, w1c,
                   b1.reshape(1, IN_FEAT), W2, b2.reshape(1, IN_FEAT),
                   W3, b3.reshape(1, 1))


# speedup vs baseline: 1.3375x; 1.0785x over previous
import functools

import jax
import jax.numpy as jnp
from jax import lax
from jax.experimental import pallas as pl
from jax.experimental.pallas import tpu as pltpu
from jax.experimental.pallas import tpu_sc as plsc

N_NODES = 10000
IN_FEAT = 128
N_EDGES = 320000

NC, NS, L = 2, 16, 16
NW = NC * NS
E_PER_W = N_EDGES // NW
CHUNK = 128
E_PAD = 10112
N_CHUNKS = E_PAD // CHUNK
ACC_ROWS = 10240
ROWS_PER_SUB = ACC_ROWS // NS
DUMP_ROW = 10016
ZBLK = 128
HE_W = 16


def _sc_aggregate(h, idx, he_w, zero_hn):
    mesh = plsc.VectorSubcoreMesh(core_axis_name="c", subcore_axis_name="s")

    @functools.partial(
        pl.kernel,
        out_type=[
            jax.ShapeDtypeStruct((NC, ACC_ROWS, IN_FEAT), jnp.float32),
            jax.ShapeDtypeStruct((NC, ACC_ROWS), jnp.float32),
        ],
        mesh=mesh,
        scratch_types=[
            pltpu.VMEM_SHARED((ACC_ROWS, IN_FEAT), jnp.float32),
            pltpu.VMEM_SHARED((ACC_ROWS,), jnp.float32),
            pltpu.VMEM((2 * CHUNK,), jnp.int32),
            pltpu.VMEM((CHUNK,), jnp.float32),
            pltpu.VMEM((CHUNK, IN_FEAT), jnp.float32),
            pltpu.VMEM((ROWS_PER_SUB,), jnp.float32),
            pltpu.SemaphoreType.DMA,
            pltpu.SemaphoreType.DMA,
            pltpu.SemaphoreType.DMA,
        ],
    )
    def agg(h_hbm, idx_hbm, he_hbm, zhn_hbm,
            hn_out_hbm, he_out_hbm,
            hn_acc, he_acc, idx_c, he_c, rows_v, he_stage, sem,
            sem_he, sem_hl):
        cid = lax.axis_index("c")
        sid = lax.axis_index("s")
        wid = cid * NS + sid
        r0 = sid * ROWS_PER_SUB

        zstage = rows_v.at[pl.ds(0, ZBLK)]
        pltpu.sync_copy(zhn_hbm, zstage)
        for k in range(ROWS_PER_SUB // ZBLK):
            pltpu.sync_copy(zstage, hn_acc.at[pl.ds(r0 + k * ZBLK, ZBLK)])
        for i in range(ROWS_PER_SUB // L):
            he_stage[pl.ds(i * L, L)] = jnp.zeros((L,), jnp.float32)
        pltpu.sync_copy(he_stage, he_acc.at[pl.ds(r0, ROWS_PER_SUB)])

        plsc.subcore_barrier()

        def body(j, carry):
            pltpu.sync_copy(
                idx_hbm.at[wid].at[pl.ds(j * 2 * CHUNK, 2 * CHUNK)], idx_c)
            d_hl = pltpu.async_copy(
                he_hbm.at[wid].at[pl.ds(j * CHUNK, CHUNK)], he_c, sem_hl)
            dst_c = idx_c.at[pl.ds(0, CHUNK)]
            src_c = idx_c.at[pl.ds(CHUNK, CHUNK)]
            pltpu.async_copy(h_hbm.at[src_c], rows_v, sem).wait()
            d_hl.wait()
            d_he = pltpu.async_copy(he_c, he_acc.at[dst_c], sem_he, add=True)
            pltpu.sync_copy(rows_v, hn_acc.at[dst_c], add=True)
            d_he.wait()
            return carry

        lax.fori_loop(0, N_CHUNKS, body, 0)

        plsc.subcore_barrier()

        for k in range(ROWS_PER_SUB // ZBLK):
            rr = r0 + k * ZBLK
            pltpu.sync_copy(hn_acc.at[pl.ds(rr, ZBLK)], zstage)
            pltpu.sync_copy(zstage, hn_out_hbm.at[cid].at[pl.ds(rr, ZBLK)])
        pltpu.sync_copy(he_acc.at[pl.ds(r0, ROWS_PER_SUB)], he_stage)
        pltpu.sync_copy(he_stage, he_out_hbm.at[cid].at[pl.ds(r0, ROWS_PER_SUB)])

    return agg(h, idx, he_w, zero_hn)


def _tc_mlp(h, hn_p, he0, he1, W1a, W1b, w1c, b1, W2, b2, W3, b3):
    NB = 10
    BLK = N_NODES // NB

    def body(h_ref, hn_ref, he0_ref, he1_ref,
             W1a_ref, W1b_ref, w1c_ref, b1_ref, W2_ref, b2_ref,
             W3_ref, b3_ref, out_ref):
        hn = hn_ref[0] + hn_ref[1]
        he_col = he0_ref[...] + he1_ref[...]
        x = jnp.dot(h_ref[...], W1a_ref[...], preferred_element_type=jnp.float32)
        x = x + jnp.dot(hn, W1b_ref[...], preferred_element_type=jnp.float32)
        x = jnp.maximum(x + he_col * w1c_ref[...] + b1_ref[...], 0.0)
        x = jnp.maximum(
            jnp.dot(x, W2_ref[...], preferred_element_type=jnp.float32) + b2_ref[...], 0.0)
        out_ref[...] = (
            jnp.dot(x, W3_ref[...], preferred_element_type=jnp.float32) + b3_ref[...])

    full = lambda i: (0, 0)
    rows = lambda i: (i, 0)
    return pl.pallas_call(
        body,
        grid=(NB,),
        in_specs=[
            pl.BlockSpec((BLK, IN_FEAT), rows),
            pl.BlockSpec((NC, BLK, IN_FEAT), lambda i: (0, i, 0)),
            pl.BlockSpec((BLK, 1), rows),
            pl.BlockSpec((BLK, 1), rows),
            pl.BlockSpec((IN_FEAT, IN_FEAT), full),
            pl.BlockSpec((IN_FEAT, IN_FEAT), full),
            pl.BlockSpec((1, IN_FEAT), full),
            pl.BlockSpec((1, IN_FEAT), full),
            pl.BlockSpec((IN_FEAT, IN_FEAT), full),
            pl.BlockSpec((1, IN_FEAT), full),
            pl.BlockSpec((IN_FEAT, 1), full),
            pl.BlockSpec((1, 1), full),
        ],
        out_specs=pl.BlockSpec((BLK, 1), rows),
        out_shape=jax.ShapeDtypeStruct((N_NODES, 1), jnp.float32),
    )(h, hn_p, he0, he1, W1a, W1b, w1c, b1, W2, b2, W3, b3)


def kernel(h, edge_index, he, W1, b1, W2, b2, W3, b3):
    ei = edge_index.astype(jnp.int32)
    pad = E_PAD - E_PER_W
    src = jnp.pad(ei[0].reshape(NW, E_PER_W), ((0, 0), (0, pad)))
    dst = jnp.pad(ei[1].reshape(NW, E_PER_W), ((0, 0), (0, pad)),
                  constant_values=DUMP_ROW)
    idx = jnp.stack(
        [dst.reshape(NW, N_CHUNKS, CHUNK), src.reshape(NW, N_CHUNKS, CHUNK)],
        axis=2).reshape(NW, 2 * E_PAD)
    he_w = jnp.pad(he.reshape(NW, E_PER_W), ((0, 0), (0, pad)))
    zero_hn = jnp.zeros((ZBLK, IN_FEAT), jnp.float32)
    hn_p, he_p = _sc_aggregate(h, idx, he_w, zero_hn)
    he0 = he_p[0, :N_NODES].reshape(N_NODES, 1)
    he1 = he_p[1, :N_NODES].reshape(N_NODES, 1)
    W1a = W1[:IN_FEAT]
    W1b = W1[IN_FEAT:2 * IN_FEAT]
    w1c = W1[2 * IN_FEAT:]
    return _tc_mlp(h, hn_p, he0, he1, W1a, W1b, w1c,
                   b1.reshape(1, IN_FEAT), W2, b2.reshape(1, IN_FEAT),
                   W3, b3.reshape(1, 1))
